# Initial kernel scaffold; baseline (speedup 1.0000x reference)
import jax
import jax.numpy as jnp
from jax.experimental import pallas as pl


def kernel(x, pos, edge_index, edge_attr, node_attr, additional_message_features, batch, W_emb, b_emb, W_msg_0, b_msg_0, W_msg_1, b_msg_1, W_o1, b_o1, W_o2, b_o2):
    raise NotImplementedError("write your pallas kernel here")


# speedup vs baseline: 3.5299x; 3.5299x over previous
import functools

import jax
import jax.numpy as jnp
from jax import lax
from jax.experimental import pallas as pl
from jax.experimental.pallas import tpu as pltpu
from jax.experimental.pallas import tpu_sc as plsc

N = 10000
E = 320000
D = 128

NC = 2
NS = 16
NW = NC * NS
EPW = E // NW
CHUNK = 80
NCHUNK = EPW // CHUNK
NP = 10240
RPS = NP // NS

_MESH = plsc.VectorSubcoreMesh(core_axis_name="c", subcore_axis_name="s")


@functools.partial(
    pl.kernel,
    out_type=jax.ShapeDtypeStruct((2 * NP, D), jnp.float32),
    mesh=_MESH,
    scratch_types=[
        pltpu.VMEM((CHUNK,), jnp.int32),
        pltpu.VMEM((CHUNK,), jnp.int32),
        pltpu.VMEM((CHUNK + 16,), jnp.float32),
        pltpu.VMEM((CHUNK, D), jnp.float32),
        pltpu.VMEM((128, D), jnp.float32),
        pltpu.VMEM_SHARED((NP, D), jnp.float32),
        pltpu.SemaphoreType.DMA,
    ],
)
def _sc_spmm(bmat, src, dst, ea, out, src_v, dst_v, ea_v, rows_v, zbuf, acc, sem):
    c = lax.axis_index("c")
    s = lax.axis_index("s")
    wid = c * NS + s

    def _zrow(i, carry):
        for j in range(D // 16):
            zbuf[i, pl.ds(j * 16, 16)] = jnp.zeros((16,), jnp.float32)
        return carry

    lax.fori_loop(0, 128, _zrow, 0)
    for k in range(RPS // 128):
        pltpu.sync_copy(zbuf, acc.at[pl.ds(s * RPS + k * 128, 128)])
    ea_v[pl.ds(CHUNK, 16)] = jnp.zeros((16,), jnp.float32)
    plsc.subcore_barrier()

    base = wid * EPW

    def _chunk(ci, carry):
        off = base + ci * CHUNK
        pltpu.sync_copy(src.at[pl.ds(off, CHUNK)], src_v)
        pltpu.sync_copy(dst.at[pl.ds(off, CHUNK)], dst_v)
        pltpu.sync_copy(ea.at[pl.ds(off, CHUNK)], ea_v.at[pl.ds(0, CHUNK)])
        pltpu.async_copy(bmat.at[src_v], rows_v, sem).wait()

        def _scale(e, carry2):
            w = ea_v[pl.ds(e, 16)][0]
            for j in range(D // 16):
                rows_v[e, pl.ds(j * 16, 16)] = rows_v[e, pl.ds(j * 16, 16)] * w
            return carry2

        lax.fori_loop(0, CHUNK, _scale, 0)
        pltpu.sync_copy(rows_v, acc.at[dst_v], add=True)
        return carry

    lax.fori_loop(0, NCHUNK, _chunk, 0)
    plsc.subcore_barrier()
    for k in range(RPS // 128):
        pltpu.sync_copy(
            acc.at[pl.ds(s * RPS + k * 128, 128)],
            out.at[pl.ds(c * NP + s * RPS + k * 128, 128)],
        )


SCW = 8

@functools.partial(
    pl.kernel,
    out_type=jax.ShapeDtypeStruct((NW * NP * SCW,), jnp.float32),
    mesh=_MESH,
    compiler_params=pltpu.CompilerParams(needs_layout_passes=False),
    scratch_types=[
        pltpu.VMEM((CHUNK,), jnp.int32),
        pltpu.VMEM((CHUNK,), jnp.float32),
        pltpu.VMEM((CHUNK,), jnp.float32),
        pltpu.VMEM((CHUNK,), jnp.float32),
        pltpu.VMEM((NP * SCW,), jnp.float32),
    ],
)
def _sc_scalars(dst, ea, amf0, amf1, out, dst_v, ea_v, a0_v, a1_v, acc):
    c = lax.axis_index("c")
    s = lax.axis_index("s")
    wid = c * NS + s
    ones16 = jnp.ones((16,), jnp.float32)

    def _z(i, carry):
        acc[pl.ds(i * 16, 16)] = jnp.zeros((16,), jnp.float32)
        return carry

    lax.fori_loop(0, NP * SCW // 16, _z, 0)

    base = wid * EPW

    def _chunk(ci, carry):
        off = base + ci * CHUNK
        pltpu.sync_copy(dst.at[pl.ds(off, CHUNK)], dst_v)
        pltpu.sync_copy(ea.at[pl.ds(off, CHUNK)], ea_v)
        pltpu.sync_copy(amf0.at[pl.ds(off, CHUNK)], a0_v)
        pltpu.sync_copy(amf1.at[pl.ds(off, CHUNK)], a1_v)
        for g in range(CHUNK // 16):
            flat = dst_v[pl.ds(g * 16, 16)] * SCW
            ea16 = ea_v[pl.ds(g * 16, 16)]
            p = ea16 * a0_v[pl.ds(g * 16, 16)]
            q = ea16 * a1_v[pl.ds(g * 16, 16)]
            plsc.addupdate_scatter(acc, [flat], ea16)
            plsc.addupdate_scatter(acc, [flat + 1], p)
            plsc.addupdate_scatter(acc, [flat + 2], q)
            plsc.addupdate_scatter(acc, [flat + 3], ones16)
        return carry

    lax.fori_loop(0, NCHUNK, _chunk, 0)
    pltpu.sync_copy(acc, out.at[pl.ds(wid * NP * SCW, NP * SCW)])


BLK = 1000
GRID = N // BLK

_row_spec = pl.BlockSpec((BLK, D), lambda i: (i, 0))
_attr_spec = pl.BlockSpec((BLK, 1), lambda i: (i, 0))
_w_spec = pl.BlockSpec((D, D), lambda i: (0, 0))
_b_spec = pl.BlockSpec((1, D), lambda i: (0, 0))
_wa_spec = pl.BlockSpec((2, D), lambda i: (0, 0))


def _silu(v):
    return v * (1.0 / (1.0 + jnp.exp(-v)))


def _tc1_body(x_ref, na_ref, we_ref, be_ref, wd_ref, ws_ref, h_ref, a_ref, b_ref):
    h = jnp.dot(x_ref[...] * na_ref[...], we_ref[...],
                preferred_element_type=jnp.float32) + be_ref[...]
    h_ref[...] = h
    a_ref[...] = jnp.dot(h, wd_ref[...], preferred_element_type=jnp.float32)
    b_ref[...] = jnp.dot(h, ws_ref[...], preferred_element_type=jnp.float32)


_tc1 = pl.pallas_call(
    _tc1_body,
    grid=(GRID,),
    in_specs=[_row_spec, _attr_spec, _w_spec, _b_spec, _w_spec, _w_spec],
    out_specs=[_row_spec, _row_spec, _row_spec],
    out_shape=[jax.ShapeDtypeStruct((N, D), jnp.float32)] * 3,
)


def _combine(h_ref, a_ref, ga_ref, gb_ref, sp, wa_ref, bm_ref):
    s = sp[:, 0:1]
    t0 = sp[:, 1:2]
    t1 = sp[:, 2:3]
    deg = sp[:, 3:4]
    agg = (s * a_ref[...] + ga_ref[...] + gb_ref[...]
           + t0 * wa_ref[0:1, :] + t1 * wa_ref[1:2, :] + deg * bm_ref[...])
    return h_ref[...] + _silu(agg)


def _tc2_body(h_ref, a_ref, ga_ref, gb_ref, spp_ref, wa_ref, bm_ref,
              wd_ref, ws_ref, h1_ref, a1_ref, b1_ref, spr_ref):
    sp = jnp.sum(spp_ref[...], axis=0)
    spr_ref[...] = sp
    h1 = _combine(h_ref, a_ref, ga_ref, gb_ref, sp, wa_ref, bm_ref)
    h1_ref[...] = h1
    a1_ref[...] = jnp.dot(h1, wd_ref[...], preferred_element_type=jnp.float32)
    b1_ref[...] = jnp.dot(h1, ws_ref[...], preferred_element_type=jnp.float32)


_spp_spec = pl.BlockSpec((NW, BLK, SCW), lambda i: (0, i, 0))
_spr_spec = pl.BlockSpec((BLK, SCW), lambda i: (i, 0))

_tc2 = pl.pallas_call(
    _tc2_body,
    grid=(GRID,),
    in_specs=[_row_spec, _row_spec, _row_spec, _row_spec, _spp_spec,
              _wa_spec, _b_spec, _w_spec, _w_spec],
    out_specs=[_row_spec, _row_spec, _row_spec, _spr_spec],
    out_shape=[jax.ShapeDtypeStruct((N, D), jnp.float32)] * 3
    + [jax.ShapeDtypeStruct((N, SCW), jnp.float32)],
)


def _tc3_body(h_ref, a_ref, ga_ref, gb_ref, spr_ref, wa_ref, bm_ref,
              na_ref, wo1_ref, bo1_ref, wo2_ref, bo2_ref, out_ref):
    h2 = _combine(h_ref, a_ref, ga_ref, gb_ref, spr_ref[...], wa_ref, bm_ref)
    z = _silu(jnp.dot(h2 * na_ref[...], wo1_ref[...],
                      preferred_element_type=jnp.float32) + bo1_ref[...])
    out_ref[...] = jnp.dot(z * na_ref[...], wo2_ref[...],
                           preferred_element_type=jnp.float32) + bo2_ref[...]


_tc3 = pl.pallas_call(
    _tc3_body,
    grid=(GRID,),
    in_specs=[_row_spec, _row_spec, _row_spec, _row_spec, _spr_spec,
              _wa_spec, _b_spec, _attr_spec, _w_spec, _b_spec, _w_spec, _b_spec],
    out_specs=_row_spec,
    out_shape=jax.ShapeDtypeStruct((N, D), jnp.float32),
)


def kernel(x, pos, edge_index, edge_attr, node_attr, additional_message_features,
           batch, W_emb, b_emb, W_msg_0, b_msg_0, W_msg_1, b_msg_1,
           W_o1, b_o1, W_o2, b_o2):
    src = edge_index[0]
    dst = edge_index[1]
    ea = edge_attr[:, 0]
    amf0 = additional_message_features[:, 0]
    amf1 = additional_message_features[:, 1]
    wd0, ws0, wa0 = W_msg_0[:D], W_msg_0[D:2 * D], W_msg_0[2 * D:]
    wd1, ws1, wa1 = W_msg_1[:D], W_msg_1[D:2 * D], W_msg_1[2 * D:]

    h0, a0, b0 = _tc1(x, node_attr, W_emb, b_emb.reshape(1, D), wd0, ws0)
    spp = _sc_scalars(dst, ea, amf0, amf1).reshape(NW, NP, SCW)[:, :N, :]
    g0 = _sc_spmm(b0, src, dst, ea)
    h1, a1, b1, spr = _tc2(h0, a0, g0[:N], g0[NP:NP + N], spp,
                           wa0, b_msg_0.reshape(1, D), wd1, ws1)
    g1 = _sc_spmm(b1, src, dst, ea)
    out = _tc3(h1, a1, g1[:N], g1[NP:NP + N], spr,
               wa1, b_msg_1.reshape(1, D), node_attr,
               W_o1, b_o1.reshape(1, D), W_o2, b_o2.reshape(1, D))
    return out


# speedup vs baseline: 4.3514x; 1.2327x over previous
import functools

import jax
import jax.numpy as jnp
from jax import lax
from jax.experimental import pallas as pl
from jax.experimental.pallas import tpu as pltpu
from jax.experimental.pallas import tpu_sc as plsc

N = 10000
E = 320000
D = 128

NC = 2
NS = 16
NW = NC * NS
EPW = E // NW
CHUNK = 80
NCHUNK = EPW // CHUNK
NP = 10240
RPS = NP // NS

_MESH = plsc.VectorSubcoreMesh(core_axis_name="c", subcore_axis_name="s")


@functools.partial(
    pl.kernel,
    out_type=jax.ShapeDtypeStruct((2 * NP, D), jnp.float32),
    mesh=_MESH,
    scratch_types=[
        pltpu.VMEM((CHUNK,), jnp.int32),
        pltpu.VMEM((CHUNK,), jnp.int32),
        pltpu.VMEM((CHUNK,), jnp.int32),
        pltpu.VMEM((CHUNK,), jnp.int32),
        pltpu.VMEM((CHUNK + 16,), jnp.float32),
        pltpu.VMEM((CHUNK + 16,), jnp.float32),
        pltpu.VMEM((CHUNK, D), jnp.float32),
        pltpu.VMEM((CHUNK, D), jnp.float32),
        pltpu.VMEM((128, D), jnp.float32),
        pltpu.VMEM_SHARED((NP, D), jnp.float32),
        pltpu.SemaphoreType.DMA,
        pltpu.SemaphoreType.DMA,
    ],
)
def _sc_spmm(bmat, src, dst, ea, out, s0, s1, d0, d1, e0, e1, r0, r1,
             zbuf, acc, sm0, sm1):
    c = lax.axis_index("c")
    s = lax.axis_index("s")
    wid = c * NS + s
    srcb, dstb, eab, rowb, semb = [s0, s1], [d0, d1], [e0, e1], [r0, r1], [sm0, sm1]

    def _zrow(i, carry):
        for j in range(D // 16):
            zbuf[i, pl.ds(j * 16, 16)] = jnp.zeros((16,), jnp.float32)
        return carry

    lax.fori_loop(0, 128, _zrow, 0)
    for k in range(RPS // 128):
        pltpu.sync_copy(zbuf, acc.at[pl.ds(s * RPS + k * 128, 128)])
    e0[pl.ds(CHUNK, 16)] = jnp.zeros((16,), jnp.float32)
    e1[pl.ds(CHUNK, 16)] = jnp.zeros((16,), jnp.float32)
    plsc.subcore_barrier()

    base = wid * EPW

    def _load(ci, b):
        off = base + ci * CHUNK
        pltpu.sync_copy(src.at[pl.ds(off, CHUNK)], srcb[b])
        pltpu.sync_copy(dst.at[pl.ds(off, CHUNK)], dstb[b])
        pltpu.sync_copy(ea.at[pl.ds(off, CHUNK)], eab[b].at[pl.ds(0, CHUNK)])
        pltpu.async_copy(bmat.at[srcb[b]], rowb[b], semb[b])

    def _proc(b):
        pltpu.make_async_copy(bmat.at[srcb[b]], rowb[b], semb[b]).wait()

        def _scale(e, carry2):
            w = eab[b][pl.ds(e, 16)][0]
            for j in range(D // 16):
                rowb[b][e, pl.ds(j * 16, 16)] = rowb[b][e, pl.ds(j * 16, 16)] * w
            return carry2

        lax.fori_loop(0, CHUNK, _scale, 0)
        pltpu.sync_copy(rowb[b], acc.at[dstb[b]], add=True)

    _load(0, 0)

    def _pair(pi, carry):
        _load(2 * pi + 1, 1)
        _proc(0)
        _load(2 * pi + 2, 0)
        _proc(1)
        return carry

    lax.fori_loop(0, (NCHUNK - 1) // 2, _pair, 0)
    _proc(0)
    plsc.subcore_barrier()
    for k in range(RPS // 128):
        pltpu.sync_copy(
            acc.at[pl.ds(s * RPS + k * 128, 128)],
            out.at[pl.ds(c * NP + s * RPS + k * 128, 128)],
        )


SCW = 8

@functools.partial(
    pl.kernel,
    out_type=jax.ShapeDtypeStruct((NW * NP * SCW,), jnp.float32),
    mesh=_MESH,
    compiler_params=pltpu.CompilerParams(needs_layout_passes=False),
    scratch_types=[
        pltpu.VMEM((CHUNK,), jnp.int32),
        pltpu.VMEM((CHUNK,), jnp.float32),
        pltpu.VMEM((CHUNK,), jnp.float32),
        pltpu.VMEM((CHUNK,), jnp.float32),
        pltpu.VMEM((NP * SCW,), jnp.float32),
    ],
)
def _sc_scalars(dst, ea, amf0, amf1, out, dst_v, ea_v, a0_v, a1_v, acc):
    c = lax.axis_index("c")
    s = lax.axis_index("s")
    wid = c * NS + s
    ones16 = jnp.ones((16,), jnp.float32)

    def _z(i, carry):
        acc[pl.ds(i * 16, 16)] = jnp.zeros((16,), jnp.float32)
        return carry

    lax.fori_loop(0, NP * SCW // 16, _z, 0)

    base = wid * EPW

    def _chunk(ci, carry):
        off = base + ci * CHUNK
        pltpu.sync_copy(dst.at[pl.ds(off, CHUNK)], dst_v)
        pltpu.sync_copy(ea.at[pl.ds(off, CHUNK)], ea_v)
        pltpu.sync_copy(amf0.at[pl.ds(off, CHUNK)], a0_v)
        pltpu.sync_copy(amf1.at[pl.ds(off, CHUNK)], a1_v)
        for g in range(CHUNK // 16):
            flat = dst_v[pl.ds(g * 16, 16)] * SCW
            ea16 = ea_v[pl.ds(g * 16, 16)]
            p = ea16 * a0_v[pl.ds(g * 16, 16)]
            q = ea16 * a1_v[pl.ds(g * 16, 16)]
            plsc.addupdate_scatter(acc, [flat], ea16)
            plsc.addupdate_scatter(acc, [flat + 1], p)
            plsc.addupdate_scatter(acc, [flat + 2], q)
            plsc.addupdate_scatter(acc, [flat + 3], ones16)
        return carry

    lax.fori_loop(0, NCHUNK, _chunk, 0)
    pltpu.sync_copy(acc, out.at[pl.ds(wid * NP * SCW, NP * SCW)])


BLK = 1000
GRID = N // BLK

_row_spec = pl.BlockSpec((BLK, D), lambda i: (i, 0))
_attr_spec = pl.BlockSpec((BLK, 1), lambda i: (i, 0))
_w_spec = pl.BlockSpec((D, D), lambda i: (0, 0))
_b_spec = pl.BlockSpec((1, D), lambda i: (0, 0))
_wa_spec = pl.BlockSpec((2, D), lambda i: (0, 0))


def _silu(v):
    return v * (1.0 / (1.0 + jnp.exp(-v)))


def _tc1_body(x_ref, na_ref, we_ref, be_ref, wd_ref, ws_ref, h_ref, a_ref, b_ref):
    h = jnp.dot(x_ref[...] * na_ref[...], we_ref[...],
                preferred_element_type=jnp.float32) + be_ref[...]
    h_ref[...] = h
    a_ref[...] = jnp.dot(h, wd_ref[...], preferred_element_type=jnp.float32)
    b_ref[...] = jnp.dot(h, ws_ref[...], preferred_element_type=jnp.float32)


_tc1 = pl.pallas_call(
    _tc1_body,
    grid=(GRID,),
    in_specs=[_row_spec, _attr_spec, _w_spec, _b_spec, _w_spec, _w_spec],
    out_specs=[_row_spec, _row_spec, _row_spec],
    out_shape=[jax.ShapeDtypeStruct((N, D), jnp.float32)] * 3,
)


def _combine(h_ref, a_ref, ga_ref, gb_ref, sp, wa_ref, bm_ref):
    s = sp[:, 0:1]
    t0 = sp[:, 1:2]
    t1 = sp[:, 2:3]
    deg = sp[:, 3:4]
    agg = (s * a_ref[...] + ga_ref[...] + gb_ref[...]
           + t0 * wa_ref[0:1, :] + t1 * wa_ref[1:2, :] + deg * bm_ref[...])
    return h_ref[...] + _silu(agg)


def _tc2_body(h_ref, a_ref, ga_ref, gb_ref, spp_ref, wa_ref, bm_ref,
              wd_ref, ws_ref, h1_ref, a1_ref, b1_ref, spr_ref):
    sp = jnp.sum(spp_ref[...], axis=0)
    spr_ref[...] = sp
    h1 = _combine(h_ref, a_ref, ga_ref, gb_ref, sp, wa_ref, bm_ref)
    h1_ref[...] = h1
    a1_ref[...] = jnp.dot(h1, wd_ref[...], preferred_element_type=jnp.float32)
    b1_ref[...] = jnp.dot(h1, ws_ref[...], preferred_element_type=jnp.float32)


_spp_spec = pl.BlockSpec((NW, BLK, SCW), lambda i: (0, i, 0))
_spr_spec = pl.BlockSpec((BLK, SCW), lambda i: (i, 0))

_tc2 = pl.pallas_call(
    _tc2_body,
    grid=(GRID,),
    in_specs=[_row_spec, _row_spec, _row_spec, _row_spec, _spp_spec,
              _wa_spec, _b_spec, _w_spec, _w_spec],
    out_specs=[_row_spec, _row_spec, _row_spec, _spr_spec],
    out_shape=[jax.ShapeDtypeStruct((N, D), jnp.float32)] * 3
    + [jax.ShapeDtypeStruct((N, SCW), jnp.float32)],
)


def _tc3_body(h_ref, a_ref, ga_ref, gb_ref, spr_ref, wa_ref, bm_ref,
              na_ref, wo1_ref, bo1_ref, wo2_ref, bo2_ref, out_ref):
    h2 = _combine(h_ref, a_ref, ga_ref, gb_ref, spr_ref[...], wa_ref, bm_ref)
    z = _silu(jnp.dot(h2 * na_ref[...], wo1_ref[...],
                      preferred_element_type=jnp.float32) + bo1_ref[...])
    out_ref[...] = jnp.dot(z * na_ref[...], wo2_ref[...],
                           preferred_element_type=jnp.float32) + bo2_ref[...]


_tc3 = pl.pallas_call(
    _tc3_body,
    grid=(GRID,),
    in_specs=[_row_spec, _row_spec, _row_spec, _row_spec, _spr_spec,
              _wa_spec, _b_spec, _attr_spec, _w_spec, _b_spec, _w_spec, _b_spec],
    out_specs=_row_spec,
    out_shape=jax.ShapeDtypeStruct((N, D), jnp.float32),
)


def kernel(x, pos, edge_index, edge_attr, node_attr, additional_message_features,
           batch, W_emb, b_emb, W_msg_0, b_msg_0, W_msg_1, b_msg_1,
           W_o1, b_o1, W_o2, b_o2):
    src = edge_index[0]
    dst = edge_index[1]
    ea = edge_attr[:, 0]
    amf0 = additional_message_features[:, 0]
    amf1 = additional_message_features[:, 1]
    wd0, ws0, wa0 = W_msg_0[:D], W_msg_0[D:2 * D], W_msg_0[2 * D:]
    wd1, ws1, wa1 = W_msg_1[:D], W_msg_1[D:2 * D], W_msg_1[2 * D:]

    h0, a0, b0 = _tc1(x, node_attr, W_emb, b_emb.reshape(1, D), wd0, ws0)
    spp = _sc_scalars(dst, ea, amf0, amf1).reshape(NW, NP, SCW)[:, :N, :]
    g0 = _sc_spmm(b0, src, dst, ea)
    h1, a1, b1, spr = _tc2(h0, a0, g0[:N], g0[NP:NP + N], spp,
                           wa0, b_msg_0.reshape(1, D), wd1, ws1)
    g1 = _sc_spmm(b1, src, dst, ea)
    out = _tc3(h1, a1, g1[:N], g1[NP:NP + N], spr,
               wa1, b_msg_1.reshape(1, D), node_attr,
               W_o1, b_o1.reshape(1, D), W_o2, b_o2.reshape(1, D))
    return out


# speedup vs baseline: 5.2704x; 1.2112x over previous
import functools

import jax
import jax.numpy as jnp
from jax import lax
from jax.experimental import pallas as pl
from jax.experimental.pallas import tpu as pltpu
from jax.experimental.pallas import tpu_sc as plsc

N = 10000
E = 320000
D = 128

NC = 2
NS = 16
NW = NC * NS
EPW = E // NW
CHUNK = 80
NCHUNK = EPW // CHUNK
NP = 10240
RPS = NP // NS

_MESH = plsc.VectorSubcoreMesh(core_axis_name="c", subcore_axis_name="s")


@functools.partial(
    pl.kernel,
    out_type=jax.ShapeDtypeStruct((2 * NP, D), jnp.float32),
    mesh=_MESH,
    scratch_types=[
        pltpu.VMEM((CHUNK,), jnp.int32),
        pltpu.VMEM((CHUNK,), jnp.int32),
        pltpu.VMEM((CHUNK,), jnp.int32),
        pltpu.VMEM((CHUNK,), jnp.int32),
        pltpu.VMEM((CHUNK + 16,), jnp.float32),
        pltpu.VMEM((CHUNK + 16,), jnp.float32),
        pltpu.VMEM((CHUNK, D), jnp.float32),
        pltpu.VMEM((CHUNK, D), jnp.float32),
        pltpu.VMEM((128, D), jnp.float32),
        pltpu.VMEM_SHARED((NP, D), jnp.float32),
        pltpu.SemaphoreType.DMA,
        pltpu.SemaphoreType.DMA,
    ],
)
def _sc_spmm(bmat, src, dst, ea, out, s0, s1, d0, d1, e0, e1, r0, r1,
             zbuf, acc, sm0, sm1):
    c = lax.axis_index("c")
    s = lax.axis_index("s")
    wid = c * NS + s
    srcb, dstb, eab, rowb, semb = [s0, s1], [d0, d1], [e0, e1], [r0, r1], [sm0, sm1]

    def _zrow(i, carry):
        for j in range(D // 16):
            zbuf[i, pl.ds(j * 16, 16)] = jnp.zeros((16,), jnp.float32)
        return carry

    lax.fori_loop(0, 128, _zrow, 0)
    for k in range(RPS // 128):
        pltpu.sync_copy(zbuf, acc.at[pl.ds(s * RPS + k * 128, 128)])
    e0[pl.ds(CHUNK, 16)] = jnp.zeros((16,), jnp.float32)
    e1[pl.ds(CHUNK, 16)] = jnp.zeros((16,), jnp.float32)
    plsc.subcore_barrier()

    base = wid * EPW

    def _load(ci, b):
        off = base + ci * CHUNK
        pltpu.sync_copy(src.at[pl.ds(off, CHUNK)], srcb[b])
        pltpu.sync_copy(dst.at[pl.ds(off, CHUNK)], dstb[b])
        pltpu.sync_copy(ea.at[pl.ds(off, CHUNK)], eab[b].at[pl.ds(0, CHUNK)])
        pltpu.async_copy(bmat.at[srcb[b]], rowb[b], semb[b])

    def _proc(b):
        pltpu.make_async_copy(bmat.at[srcb[b]], rowb[b], semb[b]).wait()

        def _scale(e, carry2):
            w = eab[b][pl.ds(e, 16)][0]
            for j in range(D // 16):
                rowb[b][e, pl.ds(j * 16, 16)] = rowb[b][e, pl.ds(j * 16, 16)] * w
            return carry2

        lax.fori_loop(0, CHUNK, _scale, 0)
        pltpu.sync_copy(rowb[b], acc.at[dstb[b]], add=True)

    _load(0, 0)

    def _pair(pi, carry):
        _load(2 * pi + 1, 1)
        _proc(0)
        _load(2 * pi + 2, 0)
        _proc(1)
        return carry

    lax.fori_loop(0, (NCHUNK - 1) // 2, _pair, 0)
    _proc(0)
    plsc.subcore_barrier()
    for k in range(RPS // 128):
        pltpu.sync_copy(
            acc.at[pl.ds(s * RPS + k * 128, 128)],
            out.at[pl.ds(c * NP + s * RPS + k * 128, 128)],
        )


SCW = 8
CHUNK_S = 2000
NCHUNK_S = EPW // CHUNK_S

@functools.partial(
    pl.kernel,
    out_type=jax.ShapeDtypeStruct((NW * NP * SCW,), jnp.float32),
    mesh=_MESH,
    compiler_params=pltpu.CompilerParams(needs_layout_passes=False),
    scratch_types=[
        pltpu.VMEM((CHUNK_S,), jnp.int32),
        pltpu.VMEM((CHUNK_S,), jnp.float32),
        pltpu.VMEM((CHUNK_S,), jnp.float32),
        pltpu.VMEM((CHUNK_S,), jnp.float32),
        pltpu.VMEM((NP * SCW,), jnp.float32),
    ],
)
def _sc_scalars(dst, ea, amf0, amf1, out, dst_v, ea_v, a0_v, a1_v, acc):
    c = lax.axis_index("c")
    s = lax.axis_index("s")
    wid = c * NS + s
    ones16 = jnp.ones((16,), jnp.float32)

    def _z(i, carry):
        acc[pl.ds(i * 16, 16)] = jnp.zeros((16,), jnp.float32)
        return carry

    lax.fori_loop(0, NP * SCW // 16, _z, 0)

    base = wid * EPW

    def _chunk(ci, carry):
        off = base + ci * CHUNK_S
        pltpu.sync_copy(dst.at[pl.ds(off, CHUNK_S)], dst_v)
        pltpu.sync_copy(ea.at[pl.ds(off, CHUNK_S)], ea_v)
        pltpu.sync_copy(amf0.at[pl.ds(off, CHUNK_S)], a0_v)
        pltpu.sync_copy(amf1.at[pl.ds(off, CHUNK_S)], a1_v)

        def _grp(g, carry2):
            flat = dst_v[pl.ds(g * 16, 16)] * SCW
            ea16 = ea_v[pl.ds(g * 16, 16)]
            p = ea16 * a0_v[pl.ds(g * 16, 16)]
            q = ea16 * a1_v[pl.ds(g * 16, 16)]
            plsc.addupdate_scatter(acc, [flat], ea16)
            plsc.addupdate_scatter(acc, [flat + 1], p)
            plsc.addupdate_scatter(acc, [flat + 2], q)
            plsc.addupdate_scatter(acc, [flat + 3], ones16)
            return carry2

        lax.fori_loop(0, CHUNK_S // 16, _grp, 0)
        return carry

    lax.fori_loop(0, NCHUNK_S, _chunk, 0)
    pltpu.sync_copy(acc, out.at[pl.ds(wid * NP * SCW, NP * SCW)])


BLK = 1000
GRID = N // BLK

_row_spec = pl.BlockSpec((BLK, D), lambda i: (i, 0))
_attr_spec = pl.BlockSpec((BLK, 1), lambda i: (i, 0))
_w_spec = pl.BlockSpec((D, D), lambda i: (0, 0))
_b_spec = pl.BlockSpec((1, D), lambda i: (0, 0))
_wa_spec = pl.BlockSpec((2, D), lambda i: (0, 0))


def _silu(v):
    return v * (1.0 / (1.0 + jnp.exp(-v)))


def _tc1_body(x_ref, na_ref, we_ref, be_ref, wd_ref, ws_ref, h_ref, a_ref, b_ref):
    h = jnp.dot(x_ref[...] * na_ref[...], we_ref[...],
                preferred_element_type=jnp.float32) + be_ref[...]
    h_ref[...] = h
    a_ref[...] = jnp.dot(h, wd_ref[...], preferred_element_type=jnp.float32)
    b_ref[...] = jnp.dot(h, ws_ref[...], preferred_element_type=jnp.float32)


_tc1 = pl.pallas_call(
    _tc1_body,
    grid=(GRID,),
    in_specs=[_row_spec, _attr_spec, _w_spec, _b_spec, _w_spec, _w_spec],
    out_specs=[_row_spec, _row_spec, _row_spec],
    out_shape=[jax.ShapeDtypeStruct((N, D), jnp.float32)] * 3,
)


def _combine(h_ref, a_ref, ga_ref, gb_ref, sp, wa_ref, bm_ref):
    s = sp[:, 0:1]
    t0 = sp[:, 1:2]
    t1 = sp[:, 2:3]
    deg = sp[:, 3:4]
    agg = (s * a_ref[...] + ga_ref[...] + gb_ref[...]
           + t0 * wa_ref[0:1, :] + t1 * wa_ref[1:2, :] + deg * bm_ref[...])
    return h_ref[...] + _silu(agg)


def _tc2_body(h_ref, a_ref, ga_ref, gb_ref, spp_ref, wa_ref, bm_ref,
              wd_ref, ws_ref, h1_ref, a1_ref, b1_ref, spr_ref):
    sp = jnp.sum(spp_ref[...], axis=0)
    spr_ref[...] = sp
    h1 = _combine(h_ref, a_ref, ga_ref, gb_ref, sp, wa_ref, bm_ref)
    h1_ref[...] = h1
    a1_ref[...] = jnp.dot(h1, wd_ref[...], preferred_element_type=jnp.float32)
    b1_ref[...] = jnp.dot(h1, ws_ref[...], preferred_element_type=jnp.float32)


_spp_spec = pl.BlockSpec((NW, BLK, SCW), lambda i: (0, i, 0))
_spr_spec = pl.BlockSpec((BLK, SCW), lambda i: (i, 0))

_tc2 = pl.pallas_call(
    _tc2_body,
    grid=(GRID,),
    in_specs=[_row_spec, _row_spec, _row_spec, _row_spec, _spp_spec,
              _wa_spec, _b_spec, _w_spec, _w_spec],
    out_specs=[_row_spec, _row_spec, _row_spec, _spr_spec],
    out_shape=[jax.ShapeDtypeStruct((N, D), jnp.float32)] * 3
    + [jax.ShapeDtypeStruct((N, SCW), jnp.float32)],
)


def _tc3_body(h_ref, a_ref, ga_ref, gb_ref, spr_ref, wa_ref, bm_ref,
              na_ref, wo1_ref, bo1_ref, wo2_ref, bo2_ref, out_ref):
    h2 = _combine(h_ref, a_ref, ga_ref, gb_ref, spr_ref[...], wa_ref, bm_ref)
    z = _silu(jnp.dot(h2 * na_ref[...], wo1_ref[...],
                      preferred_element_type=jnp.float32) + bo1_ref[...])
    out_ref[...] = jnp.dot(z * na_ref[...], wo2_ref[...],
                           preferred_element_type=jnp.float32) + bo2_ref[...]


_tc3 = pl.pallas_call(
    _tc3_body,
    grid=(GRID,),
    in_specs=[_row_spec, _row_spec, _row_spec, _row_spec, _spr_spec,
              _wa_spec, _b_spec, _attr_spec, _w_spec, _b_spec, _w_spec, _b_spec],
    out_specs=_row_spec,
    out_shape=jax.ShapeDtypeStruct((N, D), jnp.float32),
)


def kernel(x, pos, edge_index, edge_attr, node_attr, additional_message_features,
           batch, W_emb, b_emb, W_msg_0, b_msg_0, W_msg_1, b_msg_1,
           W_o1, b_o1, W_o2, b_o2):
    src = edge_index[0]
    dst = edge_index[1]
    ea = edge_attr[:, 0]
    amf0 = additional_message_features[:, 0]
    amf1 = additional_message_features[:, 1]
    wd0, ws0, wa0 = W_msg_0[:D], W_msg_0[D:2 * D], W_msg_0[2 * D:]
    wd1, ws1, wa1 = W_msg_1[:D], W_msg_1[D:2 * D], W_msg_1[2 * D:]

    h0, a0, b0 = _tc1(x, node_attr, W_emb, b_emb.reshape(1, D), wd0, ws0)
    spp = _sc_scalars(dst, ea, amf0, amf1).reshape(NW, NP, SCW)[:, :N, :]
    g0 = _sc_spmm(b0, src, dst, ea)
    h1, a1, b1, spr = _tc2(h0, a0, g0[:N], g0[NP:NP + N], spp,
                           wa0, b_msg_0.reshape(1, D), wd1, ws1)
    g1 = _sc_spmm(b1, src, dst, ea)
    out = _tc3(h1, a1, g1[:N], g1[NP:NP + N], spr,
               wa1, b_msg_1.reshape(1, D), node_attr,
               W_o1, b_o1.reshape(1, D), W_o2, b_o2.reshape(1, D))
    return out
